# baseline (speedup 1.0000x reference)
import jax
import jax.numpy as jnp
from jax import lax
from jax.experimental import pallas as pl
from jax.experimental.pallas import tpu as pltpu
from jax.experimental.pallas import tpu_sc as plsc

N_ITEMS = 100000
D = 128
BATCH = 4096
HIST = 50
HP = 56
NC = 2
NS = 16
NW = NC * NS
B_PER_W = BATCH // NW
GB = 8
N_GROUPS = B_PER_W // GB
SLAB = B_PER_W * HP
OFFS = (0, 16, 32, 48)


def _emb_body(x_hbm, tab_hbm, out_hbm, idx_v, buf_a, buf_b, sem_a, sem_b):
    wid = lax.axis_index("s") * NC + lax.axis_index("c")
    row0 = wid * B_PER_W

    pltpu.sync_copy(x_hbm.at[pl.ds(wid * SLAB, SLAB)], idx_v)

    zeros16 = jnp.zeros((16,), jnp.float32)

    def gathers(b0, buf, sem):
        return [
            pltpu.make_async_copy(
                tab_hbm.at[idx_v.at[pl.ds((b0 + i) * HP, HIST)]],
                buf.at[i, pl.ds(0, HIST)], sem)
            for i in range(GB)
        ]

    def start(b0, buf, sem):
        for c in gathers(b0, buf, sem):
            c.start()

    def drain(b0, buf, sem):
        for c in gathers(b0, buf, sem):
            c.wait()

    def fixup(buf, b0):
        def scan_batch(i, vmin):
            base = (b0 + i) * HP
            for o in OFFS:
                vmin = jnp.minimum(vmin, idx_v[pl.ds(base + o, 16)])
            return vmin

        vmin = lax.fori_loop(0, GB, scan_batch, jnp.ones((16,), jnp.int32))
        m = vmin[0]
        for l in range(1, 16):
            m = jnp.minimum(m, vmin[l])

        @pl.when(m == 0)
        def _():
            def fix_batch(i, carry):
                base = (b0 + i) * HP
                for o in OFFS:
                    v = idx_v[pl.ds(base + o, 16)]
                    for l in range(16):
                        if o + l < HIST:
                            @pl.when(v[l] == 0)
                            def _(h=o + l, i=i):
                                for cblk in range(D // 16):
                                    buf[i, h, pl.ds(cblk * 16, 16)] = zeros16
                return carry

            lax.fori_loop(0, GB, fix_batch, 0)

    start(0, buf_a, sem_a)

    def pair_body(p, carry):
        ga = 2 * p
        gb = ga + 1
        drain(ga * GB, buf_a, sem_a)
        start(gb * GB, buf_b, sem_b)
        fixup(buf_a, ga * GB)
        pltpu.sync_copy(buf_a, out_hbm.at[pl.ds(row0 + ga * GB, GB)])
        drain(gb * GB, buf_b, sem_b)

        @pl.when(p < N_GROUPS // 2 - 1)
        def _():
            start((ga + 2) * GB, buf_a, sem_a)

        fixup(buf_b, gb * GB)
        pltpu.sync_copy(buf_b, out_hbm.at[pl.ds(row0 + gb * GB, GB)])
        return carry

    lax.fori_loop(0, N_GROUPS // 2, pair_body, 0)


def kernel(X, table):
    xp = jnp.pad(X, ((0, 0), (0, HP - HIST)), constant_values=1)
    xp = xp.reshape(BATCH * HP)
    mesh = plsc.VectorSubcoreMesh(core_axis_name="c", subcore_axis_name="s")
    out = pl.kernel(
        _emb_body,
        out_type=jax.ShapeDtypeStruct((BATCH, HP, D), jnp.float32),
        mesh=mesh,
        scratch_types=[
            pltpu.VMEM((SLAB,), jnp.int32),
            pltpu.VMEM((GB, HP, D), jnp.float32),
            pltpu.VMEM((GB, HP, D), jnp.float32),
            pltpu.SemaphoreType.DMA,
            pltpu.SemaphoreType.DMA,
        ],
    )(xp, table)
    return out[:, :HIST, :]


# speedup vs baseline: 2.0411x; 2.0411x over previous
import jax
import jax.numpy as jnp
from jax import lax
from jax.experimental import pallas as pl
from jax.experimental.pallas import tpu as pltpu
from jax.experimental.pallas import tpu_sc as plsc

N_ITEMS = 100000
D = 128
BATCH = 4096
HIST = 50
NC = 2
NS = 16
NW = NC * NS
CHUNK = BATCH // NW
NBUF = 5


def _emb_body(xt_hbm, tab_hbm, out_hbm, idx_v, *rest):
    bufs = rest[:NBUF]
    gsems = rest[NBUF:2 * NBUF]
    wsems = rest[2 * NBUF:3 * NBUF]
    wid = lax.axis_index("s") * NC + lax.axis_index("c")
    col0 = wid * CHUNK

    pltpu.sync_copy(xt_hbm.at[:, pl.ds(col0, CHUNK)], idx_v)

    zeros16 = jnp.zeros((16,), jnp.float32)

    def gather(j, b):
        return pltpu.make_async_copy(
            tab_hbm.at[idx_v.at[j]], bufs[b], gsems[b])

    def write(j, b):
        return pltpu.make_async_copy(
            bufs[b], out_hbm.at[j, pl.ds(col0, CHUNK)], wsems[b])

    def fixup(b, j):
        buf = bufs[b]
        vmin = idx_v[j, pl.ds(0, 16)]
        for g in range(1, CHUNK // 16):
            vmin = jnp.minimum(vmin, idx_v[j, pl.ds(g * 16, 16)])
        m = vmin[0]
        for l in range(1, 16):
            m = jnp.minimum(m, vmin[l])

        @pl.when(m == 0)
        def _():
            def group_body(g, carry):
                iv = idx_v[j, pl.ds(g * 16, 16)]
                for l in range(16):
                    @pl.when(iv[l] == 0)
                    def _(l=l):
                        for cblk in range(D // 16):
                            buf[g * 16 + l, pl.ds(cblk * 16, 16)] = zeros16
                return carry

            lax.fori_loop(0, CHUNK // 16, group_body, 0)

    for b in range(NBUF - 2):
        gather(b, b).start()

    def round_body(k, carry):
        for b in range(NBUF):
            j = NBUF * k + b
            gather(j, b).wait()
            fixup(b, j)
            write(j, b).start()
            nj = j + NBUF - 2
            nb = (b + NBUF - 2) % NBUF

            @pl.when(nj < HIST)
            def _():
                @pl.when(j >= 2)
                def _():
                    write(nj - NBUF, nb).wait()

                gather(nj, nb).start()
        return carry

    lax.fori_loop(0, HIST // NBUF, round_body, 0)

    for b in range(NBUF):
        write(HIST - NBUF + b, b).wait()


def kernel(X, table):
    xt = jnp.transpose(X)
    mesh = plsc.VectorSubcoreMesh(core_axis_name="c", subcore_axis_name="s")
    out = pl.kernel(
        _emb_body,
        out_type=jax.ShapeDtypeStruct((HIST, BATCH, D), jnp.float32),
        mesh=mesh,
        scratch_types=[
            pltpu.VMEM((HIST, CHUNK), jnp.int32),
            *[pltpu.VMEM((CHUNK, D), jnp.float32) for _ in range(NBUF)],
            *[pltpu.SemaphoreType.DMA for _ in range(2 * NBUF)],
        ],
    )(xt, table)
    return jnp.transpose(out, (1, 0, 2))
